# baseline (speedup 1.0000x reference)
import functools

import jax
import jax.numpy as jnp
from jax import lax
from jax.experimental import pallas as pl
from jax.experimental.pallas import tpu as pltpu
from jax.experimental.pallas import tpu_sc as plsc

VOCAB = 100000
N_EMBD = 1024
CTX = 4096
BATCH = 4
SEQ = 4096

NUM_CORES = 2
NUM_SUBCORES = 16
NUM_WORKERS = NUM_CORES * NUM_SUBCORES
LANES = 16

POS_PER_W = SEQ // NUM_WORKERS
CHUNK = 8
NGRP = 3
PCHUNKS = POS_PER_W // CHUNK
CPR = N_EMBD // LANES
ADD_UNROLL = 8


def _embed_sc(ids_flat, token_table, pos_table):
    mesh = plsc.VectorSubcoreMesh(core_axis_name="c", subcore_axis_name="s")

    @functools.partial(
        pl.kernel,
        out_type=jax.ShapeDtypeStruct((BATCH * SEQ, N_EMBD), jnp.float32),
        mesh=mesh,
        scratch_types=(
            [pltpu.VMEM((BATCH * POS_PER_W,), jnp.int32)]
            + [pltpu.VMEM((CHUNK, N_EMBD), jnp.float32) for _ in range(2)]
            + [pltpu.VMEM((CHUNK, N_EMBD), jnp.float32)
               for _ in range(NGRP * BATCH)]
            + [pltpu.SemaphoreType.DMA for _ in range(3 + 2 * NGRP * BATCH)]
        ),
    )
    def k(ids_hbm, tok_hbm, pos_hbm, out_hbm, idx_v, *bufs_and_sems):
        pos_bufs = list(bufs_and_sems[:2])
        rows = list(bufs_and_sems[2:2 + NGRP * BATCH])
        sems = list(bufs_and_sems[2 + NGRP * BATCH:])
        isem, psem0, psem1 = sems[:3]
        psem = [psem0, psem1]
        gsem = sems[3:3 + NGRP * BATCH]
        osem = sems[3 + NGRP * BATCH:]

        wid = lax.axis_index("s") * NUM_CORES + lax.axis_index("c")
        pbase = wid * POS_PER_W

        pending_pos = [None, None]

        def start_pos(p):
            pending_pos[p % 2] = pltpu.async_copy(
                pos_hbm.at[pl.ds(pbase + p * CHUNK, CHUNK)],
                pos_bufs[p % 2], psem[p % 2])

        start_pos(0)
        start_pos(1)
        idx_copies = [
            pltpu.async_copy(ids_hbm.at[pl.ds(b * SEQ + pbase, POS_PER_W)],
                             idx_v.at[pl.ds(b * POS_PER_W, POS_PER_W)], isem)
            for b in range(BATCH)
        ]
        for c in idx_copies:
            c.wait()

        pending_g = [None] * (NGRP * BATCH)
        pending_o = [None] * (NGRP * BATCH)

        def start_group_gathers(g):
            slot = g % NGRP
            for b in range(BATCH):
                sb = slot * BATCH + b
                if pending_o[sb] is not None:
                    pending_o[sb].wait()
                    pending_o[sb] = None
                idx_sl = idx_v.at[pl.ds(b * POS_PER_W + g * CHUNK, CHUNK)]
                pending_g[sb] = pltpu.async_copy(
                    tok_hbm.at[idx_sl], rows[sb], gsem[sb])

        for g in range(min(NGRP - 1, PCHUNKS)):
            start_group_gathers(g)

        for g in range(PCHUNKS):
            slot = g % NGRP

            j = g + NGRP - 1
            if j < PCHUNKS:
                start_group_gathers(j)

            pending_pos[g % 2].wait()
            pending_pos[g % 2] = None
            for b in range(BATCH):
                pending_g[slot * BATCH + b].wait()
                pending_g[slot * BATCH + b] = None

            pv = pos_bufs[g % 2]
            b0, b1, b2, b3 = (rows[slot * BATCH + b] for b in range(BATCH))

            @plsc.parallel_loop(0, CHUNK * CPR, unroll=ADD_UNROLL)
            def _(t, pv=pv, b0=b0, b1=b1, b2=b2, b3=b3):
                r = t >> 6
                sl = pl.ds((t & (CPR - 1)) * LANES, LANES)
                val = pv[r, sl]
                plsc.addupdate(b0.at[r, sl], val)
                plsc.addupdate(b1.at[r, sl], val)
                plsc.addupdate(b2.at[r, sl], val)
                plsc.addupdate(b3.at[r, sl], val)

            if g + 2 < PCHUNKS:
                start_pos(g + 2)

            for b in range(BATCH):
                sb = slot * BATCH + b
                pending_o[sb] = pltpu.async_copy(
                    rows[sb],
                    out_hbm.at[pl.ds(b * SEQ + pbase + g * CHUNK, CHUNK)],
                    osem[sb])

        for sb in range(NGRP * BATCH):
            if pending_o[sb] is not None:
                pending_o[sb].wait()

    return k(ids_flat, token_table, pos_table)


@jax.jit
def kernel(token_ids, token_table, pos_table):
    ids_flat = token_ids.reshape(BATCH * SEQ)
    out = _embed_sc(ids_flat, token_table, pos_table)
    return out.reshape(BATCH, SEQ, N_EMBD)


# speedup vs baseline: 1.0285x; 1.0285x over previous
import functools

import jax
import jax.numpy as jnp
from jax import lax
from jax.experimental import pallas as pl
from jax.experimental.pallas import tpu as pltpu
from jax.experimental.pallas import tpu_sc as plsc

VOCAB = 100000
N_EMBD = 1024
CTX = 4096
BATCH = 4
SEQ = 4096

NUM_CORES = 2
NUM_SUBCORES = 16
NUM_WORKERS = NUM_CORES * NUM_SUBCORES
LANES = 16

POS_PER_W = SEQ // NUM_WORKERS
CHUNK = 8
NGRP = 3
PCHUNKS = POS_PER_W // CHUNK
CPR = N_EMBD // LANES
ADD_UNROLL = 2


def _embed_sc(ids_flat, token_table, pos_table):
    mesh = plsc.VectorSubcoreMesh(core_axis_name="c", subcore_axis_name="s")

    @functools.partial(
        pl.kernel,
        out_type=jax.ShapeDtypeStruct((BATCH * SEQ, N_EMBD), jnp.float32),
        mesh=mesh,
        scratch_types=(
            [pltpu.VMEM((BATCH * POS_PER_W,), jnp.int32)]
            + [pltpu.VMEM((CHUNK, N_EMBD), jnp.float32) for _ in range(2)]
            + [pltpu.VMEM((CHUNK, N_EMBD), jnp.float32)
               for _ in range(NGRP * BATCH)]
            + [pltpu.SemaphoreType.DMA for _ in range(3 + 2 * NGRP * BATCH)]
        ),
    )
    def k(ids_hbm, tok_hbm, pos_hbm, out_hbm, idx_v, *bufs_and_sems):
        pos_bufs = list(bufs_and_sems[:2])
        rows = list(bufs_and_sems[2:2 + NGRP * BATCH])
        sems = list(bufs_and_sems[2 + NGRP * BATCH:])
        isem, psem0, psem1 = sems[:3]
        psem = [psem0, psem1]
        gsem = sems[3:3 + NGRP * BATCH]
        osem = sems[3 + NGRP * BATCH:]

        wid = lax.axis_index("s") * NUM_CORES + lax.axis_index("c")
        pbase = wid * POS_PER_W

        pending_pos = [None, None]

        def start_pos(p):
            pending_pos[p % 2] = pltpu.async_copy(
                pos_hbm.at[pl.ds(pbase + p * CHUNK, CHUNK)],
                pos_bufs[p % 2], psem[p % 2])

        start_pos(0)
        start_pos(1)
        idx_copies = [
            pltpu.async_copy(ids_hbm.at[pl.ds(b * SEQ + pbase, POS_PER_W)],
                             idx_v.at[pl.ds(b * POS_PER_W, POS_PER_W)], isem)
            for b in range(BATCH)
        ]
        for c in idx_copies:
            c.wait()

        pending_g = [None] * (NGRP * BATCH)
        pending_o = [None] * (NGRP * BATCH)

        def start_group_gathers(g):
            slot = g % NGRP
            for b in range(BATCH):
                sb = slot * BATCH + b
                if pending_o[sb] is not None:
                    pending_o[sb].wait()
                    pending_o[sb] = None
                idx_sl = idx_v.at[pl.ds(b * POS_PER_W + g * CHUNK, CHUNK)]
                pending_g[sb] = pltpu.async_copy(
                    tok_hbm.at[idx_sl], rows[sb], gsem[sb])

        for g in range(min(NGRP - 1, PCHUNKS)):
            start_group_gathers(g)

        for g in range(PCHUNKS):
            slot = g % NGRP

            j = g + NGRP - 1
            if j < PCHUNKS:
                start_group_gathers(j)

            pending_pos[g % 2].wait()
            pending_pos[g % 2] = None
            for b in range(BATCH):
                pending_g[slot * BATCH + b].wait()
                pending_g[slot * BATCH + b] = None

            pv = pos_bufs[g % 2]
            b0, b1, b2, b3 = (rows[slot * BATCH + b] for b in range(BATCH))

            @plsc.parallel_loop(0, CHUNK * CPR, unroll=ADD_UNROLL)
            def _(t, pv=pv, b0=b0, b1=b1, b2=b2, b3=b3):
                r = t >> 6
                sl = pl.ds((t & (CPR - 1)) * LANES, LANES)
                val = pv[r, sl]
                plsc.addupdate(b0.at[r, sl], val)
                plsc.addupdate(b1.at[r, sl], val)
                plsc.addupdate(b2.at[r, sl], val)
                plsc.addupdate(b3.at[r, sl], val)

            if g + 2 < PCHUNKS:
                start_pos(g + 2)

            for b in range(BATCH):
                sb = slot * BATCH + b
                pending_o[sb] = pltpu.async_copy(
                    rows[sb],
                    out_hbm.at[pl.ds(b * SEQ + pbase + g * CHUNK, CHUNK)],
                    osem[sb])

        for sb in range(NGRP * BATCH):
            if pending_o[sb] is not None:
                pending_o[sb].wait()

    return k(ids_flat, token_table, pos_table)


@jax.jit
def kernel(token_ids, token_table, pos_table):
    ids_flat = token_ids.reshape(BATCH * SEQ)
    out = _embed_sc(ids_flat, token_table, pos_table)
    return out.reshape(BATCH, SEQ, N_EMBD)
